# baseline (speedup 1.0000x reference)
import functools

import jax
import jax.numpy as jnp
from jax import lax
from jax.experimental import pallas as pl
from jax.experimental.pallas import tpu as pltpu
from jax.experimental.pallas import tpu_sc as plsc

_N = 10000
_D = 128
_E = 320000
_B = 1024
_TAU = 0.2

_NC = 2
_NS = 16
_NW = _NC * _NS
_K = 128
_CH = 80
_EPW = _CH * _K
_NPAD = 10240
_ZR = 128
_RPS = _NPAD // _NS
_GPW = 3 * _B // _NW


@functools.cache
def _make_segment_sum():
    mesh = plsc.VectorSubcoreMesh(core_axis_name="c", subcore_axis_name="s",
                                  num_cores=_NC, num_subcores=_NS)
    return functools.partial(
        pl.kernel,
        out_type=jax.ShapeDtypeStruct((_NC, _NPAD, _D), jnp.float32),
        mesh=mesh,
        scratch_types=[
            pltpu.VMEM((2, 2, _K), jnp.int32),
            pltpu.VMEM((_K, _D), jnp.float32),
            pltpu.VMEM((_K, _D), jnp.float32),
            pltpu.VMEM_SHARED((_NPAD, _D), jnp.float32),
            pltpu.SemaphoreType.DMA,
            pltpu.SemaphoreType.DMA,
            pltpu.SemaphoreType.DMA,
            pltpu.SemaphoreType.DMA,
        ],
    )(_segment_sum_body)


def _segment_sum_body(table, idxw, zeros_hbm, out, idx_v, rows0, rows1, acc,
                      sem_g0, sem_g1, sem_i0, sem_i1):
    c = lax.axis_index("c")
    s = lax.axis_index("s")
    wid = s * _NC + c
    for t in range(_RPS // _ZR):
        pltpu.sync_copy(zeros_hbm, acc.at[pl.ds(s * _RPS + t * _ZR, _ZR)])
    pltpu.sync_copy(idxw.at[wid, 0], idx_v.at[0])
    pltpu.async_copy(table.at[idx_v.at[0, 0]], rows0, sem_g0)
    pltpu.async_copy(idxw.at[wid, 1], idx_v.at[1], sem_i1)
    plsc.subcore_barrier()

    rows = (rows0, rows1)
    sem_g = (sem_g0, sem_g1)
    sem_i = (sem_i0, sem_i1)

    def half(j, slot):
        oslot = 1 - slot

        @pl.when(j + 1 < _CH)
        def _start_next_gather():
            pltpu.make_async_copy(idxw.at[wid, j + 1], idx_v.at[oslot],
                                  sem_i[oslot]).wait()
            pltpu.async_copy(table.at[idx_v.at[oslot, 0]], rows[oslot],
                             sem_g[oslot])

        pltpu.make_async_copy(table.at[idx_v.at[slot, 0]], rows[slot],
                              sem_g[slot]).wait()
        pltpu.sync_copy(rows[slot], acc.at[idx_v.at[slot, 1]], add=True)

        @pl.when(j + 2 < _CH)
        def _start_next_idx():
            pltpu.async_copy(idxw.at[wid, j + 2], idx_v.at[slot], sem_i[slot])

    def body(g, carry):
        half(2 * g, 0)
        half(2 * g + 1, 1)
        return carry

    lax.fori_loop(0, _CH // 2, body, 0)
    plsc.subcore_barrier()
    pltpu.sync_copy(acc.at[pl.ds(s * _RPS, _RPS)],
                    out.at[c, pl.ds(s * _RPS, _RPS)])


@functools.cache
def _make_gather_rows():
    mesh = plsc.VectorSubcoreMesh(core_axis_name="c", subcore_axis_name="s",
                                  num_cores=_NC, num_subcores=_NS)
    return functools.partial(
        pl.kernel,
        out_type=jax.ShapeDtypeStruct((3 * _B, _D), jnp.float32),
        mesh=mesh,
        scratch_types=[
            pltpu.VMEM((_GPW,), jnp.int32),
            pltpu.VMEM((_GPW, _D), jnp.float32),
            pltpu.SemaphoreType.DMA,
        ],
    )(_gather_rows_body)


def _gather_rows_body(table, idx, out, idx_v, rows_v, sem):
    c = lax.axis_index("c")
    s = lax.axis_index("s")
    base = (s * _NC + c) * _GPW
    pltpu.sync_copy(idx.at[pl.ds(base, _GPW)], idx_v)
    pltpu.async_copy(table.at[idx_v], rows_v, sem).wait()
    pltpu.sync_copy(rows_v, out.at[pl.ds(base, _GPW)])


def _mlp_block(eps_ref, x_ref, pa_ref, pb_ref, w1_ref, b1_ref, w2_ref,
               b2_ref, o_ref):
    z = (1.0 + eps_ref[0]) * x_ref[...] + pa_ref[0] + pb_ref[0]
    z = jnp.maximum(
        jnp.dot(z, w1_ref[...], preferred_element_type=jnp.float32)
        + b1_ref[...], 0.0)
    z = jnp.dot(z, w2_ref[...], preferred_element_type=jnp.float32) \
        + b2_ref[...]
    o_ref[...] = jnp.maximum(z, 0.0)


def _gin_mlp(x, parts, W1, b1, W2, b2, eps):
    R = 1000
    return pl.pallas_call(
        _mlp_block,
        grid=(_N // R,),
        in_specs=[
            pl.BlockSpec(memory_space=pltpu.SMEM),
            pl.BlockSpec((R, _D), lambda i: (i, 0)),
            pl.BlockSpec((1, R, _D), lambda i: (0, i, 0)),
            pl.BlockSpec((1, R, _D), lambda i: (1, i, 0)),
            pl.BlockSpec((_D, _D), lambda i: (0, 0)),
            pl.BlockSpec((1, _D), lambda i: (0, 0)),
            pl.BlockSpec((_D, _D), lambda i: (0, 0)),
            pl.BlockSpec((1, _D), lambda i: (0, 0)),
        ],
        out_specs=pl.BlockSpec((R, _D), lambda i: (i, 0)),
        out_shape=jax.ShapeDtypeStruct((_N, _D), jnp.float32),
    )(eps.reshape(1), x, parts, parts, W1, b1.reshape(1, _D), W2,
      b2.reshape(1, _D))


def _head_block(g_ref, p1_ref, pb1_ref, p2_ref, pb2_ref, o_ref):
    z = jnp.maximum(
        jnp.dot(g_ref[...], p1_ref[...], preferred_element_type=jnp.float32)
        + pb1_ref[...], 0.0)
    z = jnp.dot(z, p2_ref[...], preferred_element_type=jnp.float32) \
        + pb2_ref[...]
    sv = z[0:_B]
    sa = z[_B:2 * _B]
    sb = z[2 * _B:3 * _B]

    def cos(u, w):
        un = jnp.sqrt(jnp.sum(u * u, axis=1, keepdims=True))
        wn = jnp.sqrt(jnp.sum(w * w, axis=1, keepdims=True))
        return jnp.sum(u * w, axis=1, keepdims=True) / jnp.maximum(
            un * wn, 1e-8)

    pos = cos(sv, sa)
    neg = cos(sv, sb)
    num = jnp.exp(pos / _TAU)
    den = num + jnp.exp(neg / _TAU)
    o_ref[0, 0] = -jnp.sum(jnp.log(num / den)) / _B


def _head(g, P1, pb1, P2, pb2):
    return pl.pallas_call(
        _head_block,
        out_shape=jax.ShapeDtypeStruct((1, 1), jnp.float32),
        out_specs=pl.BlockSpec(memory_space=pltpu.SMEM),
    )(g, P1, pb1.reshape(1, _D), P2, pb2.reshape(1, _D))


def kernel(x, edge_index, v, a, b, W1_0, b1_0, W2_0, b2_0, eps_0, W1_1,
           b1_1, W2_1, b2_1, eps_1, P1, pb1, P2, pb2):
    src = edge_index[0]
    dst = edge_index[1]
    pad = _NW * _EPW - _E
    srcw = jnp.concatenate(
        [src, jnp.zeros((pad,), jnp.int32)]).reshape(_NW, _CH, _K)
    dstw = jnp.concatenate(
        [dst, jnp.full((pad,), _N, jnp.int32)]).reshape(_NW, _CH, _K)
    idxw = jnp.stack([srcw, dstw], axis=2)
    zrows = jnp.zeros((_ZR, _D), jnp.float32)

    segsum = _make_segment_sum()
    p0 = segsum(x, idxw, zrows)
    h = _gin_mlp(x, p0, W1_0, b1_0, W2_0, b2_0, eps_0)
    p1 = segsum(h, idxw, zrows)
    emb = _gin_mlp(h, p1, W1_1, b1_1, W2_1, b2_1, eps_1)

    idx = jnp.concatenate([v, a, b])
    g = _make_gather_rows()(emb, idx)
    loss = _head(g, P1, pb1, P2, pb2)
    return loss[0, 0]


# speedup vs baseline: 3.2382x; 3.2382x over previous
import functools

import jax
import jax.numpy as jnp
from jax import lax
from jax.experimental import pallas as pl
from jax.experimental.pallas import tpu as pltpu
from jax.experimental.pallas import tpu_sc as plsc

_N = 10000
_D = 128
_E = 320000
_B = 1024
_TAU = 0.2

_NC = 2
_NS = 16
_NW = _NC * _NS
_K = 128
_CH = 80
_EPW = _CH * _K
_NPAD = 10240
_ZR = 128
_RPS = _NPAD // _NS
_GPW = 3 * _B // _NW


@functools.cache
def _make_segment_sum():
    mesh = plsc.VectorSubcoreMesh(core_axis_name="c", subcore_axis_name="s",
                                  num_cores=_NC, num_subcores=_NS)
    return functools.partial(
        pl.kernel,
        out_type=jax.ShapeDtypeStruct((_NC, _NPAD, _D), jnp.float32),
        mesh=mesh,
        scratch_types=[
            pltpu.VMEM((2, 2, _K), jnp.int32),
            pltpu.VMEM((_K, _D), jnp.float32),
            pltpu.VMEM((_K, _D), jnp.float32),
            pltpu.VMEM_SHARED((_NPAD, _D), jnp.float32),
            pltpu.SemaphoreType.DMA,
            pltpu.SemaphoreType.DMA,
            pltpu.SemaphoreType.DMA,
            pltpu.SemaphoreType.DMA,
        ],
    )(_segment_sum_body)


def _segment_sum_body(table, idxw, zeros_hbm, out, idx_v, rows0, rows1, acc,
                      sem_g0, sem_g1, sem_i0, sem_i1):
    c = lax.axis_index("c")
    s = lax.axis_index("s")
    wid = s * _NC + c
    for t in range(_RPS // _ZR):
        pltpu.sync_copy(zeros_hbm, acc.at[pl.ds(s * _RPS + t * _ZR, _ZR)])
    pltpu.sync_copy(idxw.at[wid, 0], idx_v.at[0])
    pltpu.async_copy(table.at[idx_v.at[0, 0]], rows0, sem_g0)
    pltpu.async_copy(idxw.at[wid, 1], idx_v.at[1], sem_i1)
    plsc.subcore_barrier()

    rows = (rows0, rows1)
    sem_g = (sem_g0, sem_g1)
    sem_i = (sem_i0, sem_i1)

    def half(j, slot):
        oslot = 1 - slot

        @pl.when(j + 1 < _CH)
        def _start_next_gather():
            pltpu.make_async_copy(idxw.at[wid, j + 1], idx_v.at[oslot],
                                  sem_i[oslot]).wait()
            pltpu.async_copy(table.at[idx_v.at[oslot, 0]], rows[oslot],
                             sem_g[oslot])

        pltpu.make_async_copy(table.at[idx_v.at[slot, 0]], rows[slot],
                              sem_g[slot]).wait()
        pltpu.sync_copy(rows[slot], acc.at[idx_v.at[slot, 1]], add=True)

        @pl.when(j + 2 < _CH)
        def _start_next_idx():
            pltpu.async_copy(idxw.at[wid, j + 2], idx_v.at[slot], sem_i[slot])

    def body(g, carry):
        half(2 * g, 0)
        half(2 * g + 1, 1)
        return carry

    lax.fori_loop(0, _CH // 2, body, 0)
    plsc.subcore_barrier()
    pltpu.sync_copy(acc.at[pl.ds(s * _RPS, _RPS)],
                    out.at[c, pl.ds(s * _RPS, _RPS)])


@functools.cache
def _make_gather_rows():
    mesh = plsc.VectorSubcoreMesh(core_axis_name="c", subcore_axis_name="s",
                                  num_cores=_NC, num_subcores=_NS)
    return functools.partial(
        pl.kernel,
        out_type=jax.ShapeDtypeStruct((3 * _B, _D), jnp.float32),
        mesh=mesh,
        scratch_types=[
            pltpu.VMEM((_GPW,), jnp.int32),
            pltpu.VMEM((_GPW, _D), jnp.float32),
            pltpu.SemaphoreType.DMA,
        ],
    )(_gather_rows_body)


def _gather_rows_body(table, idx, out, idx_v, rows_v, sem):
    c = lax.axis_index("c")
    s = lax.axis_index("s")
    base = (s * _NC + c) * _GPW
    pltpu.sync_copy(idx.at[pl.ds(base, _GPW)], idx_v)
    pltpu.async_copy(table.at[idx_v], rows_v, sem).wait()
    pltpu.sync_copy(rows_v, out.at[pl.ds(base, _GPW)])


def _mlp_block(eps_ref, x_ref, pa_ref, pb_ref, w1_ref, b1_ref, w2_ref,
               b2_ref, o_ref):
    z = (1.0 + eps_ref[0]) * x_ref[...] + pa_ref[0] + pb_ref[0]
    z = jnp.maximum(
        jnp.dot(z, w1_ref[...], preferred_element_type=jnp.float32)
        + b1_ref[...], 0.0)
    z = jnp.dot(z, w2_ref[...], preferred_element_type=jnp.float32) \
        + b2_ref[...]
    o_ref[...] = jnp.maximum(z, 0.0)


def _gin_mlp(x, parts, W1, b1, W2, b2, eps):
    R = 1000
    return pl.pallas_call(
        _mlp_block,
        grid=(_N // R,),
        in_specs=[
            pl.BlockSpec(memory_space=pltpu.SMEM),
            pl.BlockSpec((R, _D), lambda i: (i, 0)),
            pl.BlockSpec((1, R, _D), lambda i: (0, i, 0)),
            pl.BlockSpec((1, R, _D), lambda i: (1, i, 0)),
            pl.BlockSpec((_D, _D), lambda i: (0, 0)),
            pl.BlockSpec((1, _D), lambda i: (0, 0)),
            pl.BlockSpec((_D, _D), lambda i: (0, 0)),
            pl.BlockSpec((1, _D), lambda i: (0, 0)),
        ],
        out_specs=pl.BlockSpec((R, _D), lambda i: (i, 0)),
        out_shape=jax.ShapeDtypeStruct((_N, _D), jnp.float32),
    )(eps.reshape(1), x, parts, parts, W1, b1.reshape(1, _D), W2,
      b2.reshape(1, _D))


def _head_block(g_ref, p1_ref, pb1_ref, p2_ref, pb2_ref, o_ref):
    z = jnp.maximum(
        jnp.dot(g_ref[...], p1_ref[...], preferred_element_type=jnp.float32)
        + pb1_ref[...], 0.0)
    z = jnp.dot(z, p2_ref[...], preferred_element_type=jnp.float32) \
        + pb2_ref[...]
    sv = z[0:_B]
    sa = z[_B:2 * _B]
    sb = z[2 * _B:3 * _B]

    def cos(u, w):
        un = jnp.sqrt(jnp.sum(u * u, axis=1, keepdims=True))
        wn = jnp.sqrt(jnp.sum(w * w, axis=1, keepdims=True))
        return jnp.sum(u * w, axis=1, keepdims=True) / jnp.maximum(
            un * wn, 1e-8)

    pos = cos(sv, sa)
    neg = cos(sv, sb)
    num = jnp.exp(pos / _TAU)
    den = num + jnp.exp(neg / _TAU)
    o_ref[0, 0] = -jnp.sum(jnp.log(num / den)) / _B


def _head(g, P1, pb1, P2, pb2):
    return pl.pallas_call(
        _head_block,
        out_shape=jax.ShapeDtypeStruct((1, 1), jnp.float32),
        out_specs=pl.BlockSpec(memory_space=pltpu.SMEM),
    )(g, P1, pb1.reshape(1, _D), P2, pb2.reshape(1, _D))


def kernel(x, edge_index, v, a, b, W1_0, b1_0, W2_0, b2_0, eps_0, W1_1,
           b1_1, W2_1, b2_1, eps_1, P1, pb1, P2, pb2):
    src = edge_index[0]
    dst = edge_index[1]
    pad = _NW * _EPW - _E
    pad_iota = jnp.arange(pad, dtype=jnp.int32)
    srcw = jnp.concatenate(
        [src, pad_iota % _N]).reshape(_NW, _CH, _K)
    dstw = jnp.concatenate(
        [dst, _N + pad_iota % (_NPAD - _N)]).reshape(_NW, _CH, _K)
    idxw = jnp.stack([srcw, dstw], axis=2)
    zrows = jnp.zeros((_ZR, _D), jnp.float32)

    segsum = _make_segment_sum()
    p0 = segsum(x, idxw, zrows)
    h = _gin_mlp(x, p0, W1_0, b1_0, W2_0, b2_0, eps_0)
    p1 = segsum(h, idxw, zrows)
    emb = _gin_mlp(h, p1, W1_1, b1_1, W2_1, b2_1, eps_1)

    idx = jnp.concatenate([v, a, b])
    g = _make_gather_rows()(emb, idx)
    loss = _head(g, P1, pb1, P2, pb2)
    return loss[0, 0]


# speedup vs baseline: 3.5992x; 1.1115x over previous
import functools

import jax
import jax.numpy as jnp
from jax import lax
from jax.experimental import pallas as pl
from jax.experimental.pallas import tpu as pltpu
from jax.experimental.pallas import tpu_sc as plsc

_N = 10000
_D = 128
_E = 320000
_B = 1024
_TAU = 0.2

_NC = 2
_NS = 16
_NW = _NC * _NS
_K = 128
_CH = 80
_EPW = _CH * _K
_NPAD = 10240
_ZR = 128
_RPS = _NPAD // _NS
_GPW = 3 * _B // _NW


@functools.cache
def _make_segment_sum():
    mesh = plsc.VectorSubcoreMesh(core_axis_name="c", subcore_axis_name="s",
                                  num_cores=_NC, num_subcores=_NS)
    return functools.partial(
        pl.kernel,
        out_type=jax.ShapeDtypeStruct((_NC, _NPAD, _D), jnp.float32),
        mesh=mesh,
        scratch_types=[
            pltpu.VMEM((4, 2, _K), jnp.int32),
            pltpu.VMEM((_K, _D), jnp.float32),
            pltpu.VMEM((_K, _D), jnp.float32),
            pltpu.VMEM_SHARED((_NPAD, _D), jnp.float32),
            pltpu.SemaphoreType.DMA,
            pltpu.SemaphoreType.DMA,
            pltpu.SemaphoreType.DMA,
            pltpu.SemaphoreType.DMA,
        ],
    )(_segment_sum_body)


def _segment_sum_body(table, idxw, zeros_hbm, out, idx_v, rows0, rows1, acc,
                      sem_g0, sem_g1, sem_s, sem_i):
    c = lax.axis_index("c")
    s = lax.axis_index("s")
    wid = s * _NC + c
    pltpu.sync_copy(idxw.at[wid, 0], idx_v.at[0])
    pltpu.async_copy(table.at[idx_v.at[0, 0]], rows0, sem_g0)
    pltpu.async_copy(idxw.at[wid, 1], idx_v.at[1], sem_i)
    for t in range(_RPS // _ZR):
        pltpu.sync_copy(zeros_hbm, acc.at[pl.ds(s * _RPS + t * _ZR, _ZR)])
    plsc.subcore_barrier()

    rows = (rows0, rows1)
    sem_g = (sem_g0, sem_g1)

    def step(j, rslot, islot):
        orslot = 1 - rslot
        nislot = (islot + 1) % 4

        @pl.when(j >= 1)
        def _drain_prev_scatter():
            pltpu.make_async_copy(
                rows[orslot], acc.at[idx_v.at[(islot + 3) % 4, 1]],
                sem_s).wait()

        @pl.when(j + 1 < _CH)
        def _start_next_gather():
            pltpu.make_async_copy(idxw.at[wid, j + 1], idx_v.at[nislot],
                                  sem_i).wait()
            pltpu.async_copy(table.at[idx_v.at[nislot, 0]], rows[orslot],
                             sem_g[orslot])

        pltpu.make_async_copy(table.at[idx_v.at[islot, 0]], rows[rslot],
                              sem_g[rslot]).wait()
        pltpu.async_copy(rows[rslot], acc.at[idx_v.at[islot, 1]], sem_s,
                         add=True)

        @pl.when(j + 2 < _CH)
        def _start_next_idx():
            pltpu.async_copy(idxw.at[wid, j + 2], idx_v.at[(islot + 2) % 4],
                             sem_i)

    def body(g, carry):
        for u in range(4):
            step(4 * g + u, u % 2, u)
        return carry

    lax.fori_loop(0, _CH // 4, body, 0)
    pltpu.make_async_copy(rows[1], acc.at[idx_v.at[3, 1]], sem_s).wait()
    plsc.subcore_barrier()
    pltpu.sync_copy(acc.at[pl.ds(s * _RPS, _RPS)],
                    out.at[c, pl.ds(s * _RPS, _RPS)])


@functools.cache
def _make_gather_rows():
    mesh = plsc.VectorSubcoreMesh(core_axis_name="c", subcore_axis_name="s",
                                  num_cores=_NC, num_subcores=_NS)
    return functools.partial(
        pl.kernel,
        out_type=jax.ShapeDtypeStruct((3 * _B, _D), jnp.float32),
        mesh=mesh,
        scratch_types=[
            pltpu.VMEM((_GPW,), jnp.int32),
            pltpu.VMEM((_GPW, _D), jnp.float32),
            pltpu.SemaphoreType.DMA,
        ],
    )(_gather_rows_body)


def _gather_rows_body(table, idx, out, idx_v, rows_v, sem):
    c = lax.axis_index("c")
    s = lax.axis_index("s")
    base = (s * _NC + c) * _GPW
    pltpu.sync_copy(idx.at[pl.ds(base, _GPW)], idx_v)
    pltpu.async_copy(table.at[idx_v], rows_v, sem).wait()
    pltpu.sync_copy(rows_v, out.at[pl.ds(base, _GPW)])


def _mlp_block(eps_ref, x_ref, pa_ref, pb_ref, w1_ref, b1_ref, w2_ref,
               b2_ref, o_ref):
    z = (1.0 + eps_ref[0]) * x_ref[...] + pa_ref[0] + pb_ref[0]
    z = jnp.maximum(
        jnp.dot(z, w1_ref[...], preferred_element_type=jnp.float32)
        + b1_ref[...], 0.0)
    z = jnp.dot(z, w2_ref[...], preferred_element_type=jnp.float32) \
        + b2_ref[...]
    o_ref[...] = jnp.maximum(z, 0.0)


def _gin_mlp(x, parts, W1, b1, W2, b2, eps):
    R = 1000
    return pl.pallas_call(
        _mlp_block,
        grid=(_N // R,),
        in_specs=[
            pl.BlockSpec(memory_space=pltpu.SMEM),
            pl.BlockSpec((R, _D), lambda i: (i, 0)),
            pl.BlockSpec((1, R, _D), lambda i: (0, i, 0)),
            pl.BlockSpec((1, R, _D), lambda i: (1, i, 0)),
            pl.BlockSpec((_D, _D), lambda i: (0, 0)),
            pl.BlockSpec((1, _D), lambda i: (0, 0)),
            pl.BlockSpec((_D, _D), lambda i: (0, 0)),
            pl.BlockSpec((1, _D), lambda i: (0, 0)),
        ],
        out_specs=pl.BlockSpec((R, _D), lambda i: (i, 0)),
        out_shape=jax.ShapeDtypeStruct((_N, _D), jnp.float32),
    )(eps.reshape(1), x, parts, parts, W1, b1.reshape(1, _D), W2,
      b2.reshape(1, _D))


def _head_block(g_ref, p1_ref, pb1_ref, p2_ref, pb2_ref, o_ref):
    z = jnp.maximum(
        jnp.dot(g_ref[...], p1_ref[...], preferred_element_type=jnp.float32)
        + pb1_ref[...], 0.0)
    z = jnp.dot(z, p2_ref[...], preferred_element_type=jnp.float32) \
        + pb2_ref[...]
    sv = z[0:_B]
    sa = z[_B:2 * _B]
    sb = z[2 * _B:3 * _B]

    def cos(u, w):
        un = jnp.sqrt(jnp.sum(u * u, axis=1, keepdims=True))
        wn = jnp.sqrt(jnp.sum(w * w, axis=1, keepdims=True))
        return jnp.sum(u * w, axis=1, keepdims=True) / jnp.maximum(
            un * wn, 1e-8)

    pos = cos(sv, sa)
    neg = cos(sv, sb)
    num = jnp.exp(pos / _TAU)
    den = num + jnp.exp(neg / _TAU)
    o_ref[0, 0] = -jnp.sum(jnp.log(num / den)) / _B


def _head(g, P1, pb1, P2, pb2):
    return pl.pallas_call(
        _head_block,
        out_shape=jax.ShapeDtypeStruct((1, 1), jnp.float32),
        out_specs=pl.BlockSpec(memory_space=pltpu.SMEM),
    )(g, P1, pb1.reshape(1, _D), P2, pb2.reshape(1, _D))


def kernel(x, edge_index, v, a, b, W1_0, b1_0, W2_0, b2_0, eps_0, W1_1,
           b1_1, W2_1, b2_1, eps_1, P1, pb1, P2, pb2):
    src = edge_index[0]
    dst = edge_index[1]
    pad = _NW * _EPW - _E
    pad_iota = jnp.arange(pad, dtype=jnp.int32)
    srcw = jnp.concatenate(
        [src, pad_iota % _N]).reshape(_NW, _CH, _K)
    dstw = jnp.concatenate(
        [dst, _N + pad_iota % (_NPAD - _N)]).reshape(_NW, _CH, _K)
    idxw = jnp.stack([srcw, dstw], axis=2)
    zrows = jnp.zeros((_ZR, _D), jnp.float32)

    segsum = _make_segment_sum()
    p0 = segsum(x, idxw, zrows)
    h = _gin_mlp(x, p0, W1_0, b1_0, W2_0, b2_0, eps_0)
    p1 = segsum(h, idxw, zrows)
    emb = _gin_mlp(h, p1, W1_1, b1_1, W2_1, b2_1, eps_1)

    idx = jnp.concatenate([v, a, b])
    g = _make_gather_rows()(emb, idx)
    loss = _head(g, P1, pb1, P2, pb2)
    return loss[0, 0]


# speedup vs baseline: 3.6506x; 1.0143x over previous
import functools

import jax
import jax.numpy as jnp
from jax import lax
from jax.experimental import pallas as pl
from jax.experimental.pallas import tpu as pltpu
from jax.experimental.pallas import tpu_sc as plsc

_N = 10000
_D = 128
_E = 320000
_B = 1024
_TAU = 0.2

_NC = 2
_NS = 16
_NW = _NC * _NS
_K = 128
_CH = 80
_EPW = _CH * _K
_NPAD = 10240
_ZR = 128
_RPS = _NPAD // _NS
_GPW = 3 * _B // _NW


@functools.cache
def _make_segment_sum():
    mesh = plsc.VectorSubcoreMesh(core_axis_name="c", subcore_axis_name="s",
                                  num_cores=_NC, num_subcores=_NS)
    return functools.partial(
        pl.kernel,
        out_type=jax.ShapeDtypeStruct((_NC, _NPAD, _D), jnp.float32),
        mesh=mesh,
        scratch_types=[
            pltpu.VMEM((_CH, _K), jnp.int32),
            pltpu.VMEM((4, _K), jnp.int32),
            pltpu.VMEM((_K, _D), jnp.float32),
            pltpu.VMEM((_K, _D), jnp.float32),
            pltpu.VMEM_SHARED((_NPAD, _D), jnp.float32),
            pltpu.SemaphoreType.DMA,
            pltpu.SemaphoreType.DMA,
            pltpu.SemaphoreType.DMA,
            pltpu.SemaphoreType.DMA,
            pltpu.SemaphoreType.DMA,
        ],
    )(_segment_sum_body)


def _segment_sum_body(table, srcw, dstw, zeros_hbm, out, src_v, dring,
                      rows0, rows1, acc, sem_g0, sem_g1, sem_s, sem_i0,
                      sem_i1):
    c = lax.axis_index("c")
    s = lax.axis_index("s")
    wid = s * _NC + c
    pltpu.sync_copy(srcw.at[wid], src_v)
    pltpu.sync_copy(dstw.at[wid, 0], dring.at[0])
    pltpu.async_copy(table.at[src_v.at[0]], rows0, sem_g0)
    pltpu.async_copy(dstw.at[wid, 1], dring.at[1], sem_i1)
    for t in range(_RPS // _ZR):
        pltpu.async_copy(zeros_hbm, acc.at[pl.ds(s * _RPS + t * _ZR, _ZR)],
                         sem_s)
    for t in range(_RPS // _ZR):
        pltpu.make_async_copy(
            zeros_hbm, acc.at[pl.ds(s * _RPS + t * _ZR, _ZR)], sem_s).wait()
    plsc.subcore_barrier()

    rows = (rows0, rows1)
    sem_g = (sem_g0, sem_g1)
    sem_i = (sem_i0, sem_i1)

    def step(j, rslot, dslot):
        orslot = 1 - rslot

        @pl.when(j >= 1)
        def _drain_prev_scatter():
            pltpu.make_async_copy(
                rows[orslot], acc.at[dring.at[(dslot + 3) % 4]],
                sem_s).wait()

        @pl.when(j + 1 < _CH)
        def _start_next_gather():
            pltpu.async_copy(table.at[src_v.at[j + 1]], rows[orslot],
                             sem_g[orslot])

        pltpu.make_async_copy(table.at[src_v.at[j]], rows[rslot],
                              sem_g[rslot]).wait()

        @pl.when(j >= 1)
        def _wait_dst():
            pltpu.make_async_copy(dstw.at[wid, j], dring.at[dslot],
                                  sem_i[rslot]).wait()

        pltpu.async_copy(rows[rslot], acc.at[dring.at[dslot]], sem_s,
                         add=True)

        @pl.when(j + 2 < _CH)
        def _start_next_dst():
            pltpu.async_copy(dstw.at[wid, j + 2], dring.at[(dslot + 2) % 4],
                             sem_i[rslot])

    def body(g, carry):
        for u in range(4):
            step(4 * g + u, u % 2, u)
        return carry

    lax.fori_loop(0, _CH // 4, body, 0)
    pltpu.make_async_copy(rows[1], acc.at[dring.at[3]], sem_s).wait()
    plsc.subcore_barrier()
    pltpu.sync_copy(acc.at[pl.ds(s * _RPS, _RPS)],
                    out.at[c, pl.ds(s * _RPS, _RPS)])


@functools.cache
def _make_gather_rows():
    mesh = plsc.VectorSubcoreMesh(core_axis_name="c", subcore_axis_name="s",
                                  num_cores=_NC, num_subcores=_NS)
    return functools.partial(
        pl.kernel,
        out_type=jax.ShapeDtypeStruct((3 * _B, _D), jnp.float32),
        mesh=mesh,
        scratch_types=[
            pltpu.VMEM((_GPW,), jnp.int32),
            pltpu.VMEM((_GPW, _D), jnp.float32),
            pltpu.SemaphoreType.DMA,
        ],
    )(_gather_rows_body)


def _gather_rows_body(table, idx, out, idx_v, rows_v, sem):
    c = lax.axis_index("c")
    s = lax.axis_index("s")
    base = (s * _NC + c) * _GPW
    pltpu.sync_copy(idx.at[pl.ds(base, _GPW)], idx_v)
    pltpu.async_copy(table.at[idx_v], rows_v, sem).wait()
    pltpu.sync_copy(rows_v, out.at[pl.ds(base, _GPW)])


def _mlp_block(eps_ref, x_ref, pa_ref, pb_ref, w1_ref, b1_ref, w2_ref,
               b2_ref, o_ref):
    z = (1.0 + eps_ref[0]) * x_ref[...] + pa_ref[0] + pb_ref[0]
    z = jnp.maximum(
        jnp.dot(z, w1_ref[...], preferred_element_type=jnp.float32)
        + b1_ref[...], 0.0)
    z = jnp.dot(z, w2_ref[...], preferred_element_type=jnp.float32) \
        + b2_ref[...]
    o_ref[...] = jnp.maximum(z, 0.0)


def _gin_mlp(x, parts, W1, b1, W2, b2, eps):
    R = 1000
    return pl.pallas_call(
        _mlp_block,
        grid=(_N // R,),
        in_specs=[
            pl.BlockSpec(memory_space=pltpu.SMEM),
            pl.BlockSpec((R, _D), lambda i: (i, 0)),
            pl.BlockSpec((1, R, _D), lambda i: (0, i, 0)),
            pl.BlockSpec((1, R, _D), lambda i: (1, i, 0)),
            pl.BlockSpec((_D, _D), lambda i: (0, 0)),
            pl.BlockSpec((1, _D), lambda i: (0, 0)),
            pl.BlockSpec((_D, _D), lambda i: (0, 0)),
            pl.BlockSpec((1, _D), lambda i: (0, 0)),
        ],
        out_specs=pl.BlockSpec((R, _D), lambda i: (i, 0)),
        out_shape=jax.ShapeDtypeStruct((_N, _D), jnp.float32),
    )(eps.reshape(1), x, parts, parts, W1, b1.reshape(1, _D), W2,
      b2.reshape(1, _D))


def _head_block(g_ref, p1_ref, pb1_ref, p2_ref, pb2_ref, o_ref):
    z = jnp.maximum(
        jnp.dot(g_ref[...], p1_ref[...], preferred_element_type=jnp.float32)
        + pb1_ref[...], 0.0)
    z = jnp.dot(z, p2_ref[...], preferred_element_type=jnp.float32) \
        + pb2_ref[...]
    sv = z[0:_B]
    sa = z[_B:2 * _B]
    sb = z[2 * _B:3 * _B]

    def cos(u, w):
        un = jnp.sqrt(jnp.sum(u * u, axis=1, keepdims=True))
        wn = jnp.sqrt(jnp.sum(w * w, axis=1, keepdims=True))
        return jnp.sum(u * w, axis=1, keepdims=True) / jnp.maximum(
            un * wn, 1e-8)

    pos = cos(sv, sa)
    neg = cos(sv, sb)
    num = jnp.exp(pos / _TAU)
    den = num + jnp.exp(neg / _TAU)
    o_ref[0, 0] = -jnp.sum(jnp.log(num / den)) / _B


def _head(g, P1, pb1, P2, pb2):
    return pl.pallas_call(
        _head_block,
        out_shape=jax.ShapeDtypeStruct((1, 1), jnp.float32),
        out_specs=pl.BlockSpec(memory_space=pltpu.SMEM),
    )(g, P1, pb1.reshape(1, _D), P2, pb2.reshape(1, _D))


def kernel(x, edge_index, v, a, b, W1_0, b1_0, W2_0, b2_0, eps_0, W1_1,
           b1_1, W2_1, b2_1, eps_1, P1, pb1, P2, pb2):
    src = edge_index[0]
    dst = edge_index[1]
    pad = _NW * _EPW - _E
    pad_iota = jnp.arange(pad, dtype=jnp.int32)
    srcw = jnp.concatenate(
        [src, pad_iota % _N]).reshape(_NW, _CH, _K)
    dstw = jnp.concatenate(
        [dst, _N + pad_iota % (_NPAD - _N)]).reshape(_NW, _CH, _K)
    zrows = jnp.zeros((_ZR, _D), jnp.float32)

    segsum = _make_segment_sum()
    p0 = segsum(x, srcw, dstw, zrows)
    h = _gin_mlp(x, p0, W1_0, b1_0, W2_0, b2_0, eps_0)
    p1 = segsum(h, srcw, dstw, zrows)
    emb = _gin_mlp(h, p1, W1_1, b1_1, W2_1, b2_1, eps_1)

    idx = jnp.concatenate([v, a, b])
    g = _make_gather_rows()(emb, idx)
    loss = _head(g, P1, pb1, P2, pb2)
    return loss[0, 0]


# speedup vs baseline: 3.7998x; 1.0409x over previous
import functools

import jax
import jax.numpy as jnp
import numpy as np
from jax import lax
from jax.experimental import pallas as pl
from jax.experimental.pallas import tpu as pltpu
from jax.experimental.pallas import tpu_sc as plsc

_N = 10000
_D = 128
_E = 320000
_B = 1024
_TAU = 0.2

_NC = 2
_NS = 16
_NW = _NC * _NS
_K = 128
_CH = 80
_EPW = _CH * _K
_NPAD = 10240
_ZR = 128
_RPS = _NPAD // _NS
_GPW = 3 * _B // _NW
_PAD = _NW * _EPW - _E

_PAD_EDGES = np.stack([
    np.arange(_PAD, dtype=np.int32) % _N,
    _N + np.arange(_PAD, dtype=np.int32) % (_NPAD - _N),
])


@functools.cache
def _make_segment_sum():
    mesh = plsc.VectorSubcoreMesh(core_axis_name="c", subcore_axis_name="s",
                                  num_cores=_NC, num_subcores=_NS)
    return functools.partial(
        pl.kernel,
        out_type=jax.ShapeDtypeStruct((_NC, _NPAD, _D), jnp.float32),
        mesh=mesh,
        scratch_types=[
            pltpu.VMEM((_EPW,), jnp.int32),
            pltpu.VMEM((4, _K), jnp.int32),
            pltpu.VMEM((_K, _D), jnp.float32),
            pltpu.VMEM((_K, _D), jnp.float32),
            pltpu.VMEM_SHARED((_NPAD, _D), jnp.float32),
            pltpu.SemaphoreType.DMA,
            pltpu.SemaphoreType.DMA,
            pltpu.SemaphoreType.DMA,
            pltpu.SemaphoreType.DMA,
            pltpu.SemaphoreType.DMA,
        ],
    )(_segment_sum_body)


def _segment_sum_body(table, edges, pads, zeros_hbm, out, src_v, dring,
                      rows0, rows1, acc, sem_g0, sem_g1, sem_s, sem_i0,
                      sem_i1):
    c = lax.axis_index("c")
    s = lax.axis_index("s")
    wid = s * _NC + c
    base = wid * _EPW
    real31 = _E - (_NW - 1) * _EPW

    def load_dst(j, slot, sem):
        off = base + j * _K

        @pl.when(off + _K <= _E)
        def _real():
            pltpu.async_copy(edges.at[1, pl.ds(off, _K)], dring.at[slot],
                             sem)

        @pl.when(off >= _E)
        def _padded():
            pltpu.async_copy(pads.at[1, pl.ds(off - _E, _K)],
                             dring.at[slot], sem)

    @pl.when(wid < _NW - 1)
    def _src_full():
        pltpu.sync_copy(edges.at[0, pl.ds(base, _EPW)], src_v)

    @pl.when(wid == _NW - 1)
    def _src_split():
        pltpu.sync_copy(edges.at[0, pl.ds(base, real31)],
                        src_v.at[pl.ds(0, real31)])
        pltpu.sync_copy(pads.at[0], src_v.at[pl.ds(real31, _PAD)])

    load_dst(0, 0, sem_i0)
    pltpu.make_async_copy(edges.at[1, pl.ds(base, _K)], dring.at[0],
                          sem_i0).wait()
    pltpu.async_copy(table.at[src_v.at[pl.ds(0, _K)]], rows0, sem_g0)
    load_dst(1, 1, sem_i1)
    for t in range(_RPS // _ZR):
        pltpu.async_copy(zeros_hbm, acc.at[pl.ds(s * _RPS + t * _ZR, _ZR)],
                         sem_s)
    for t in range(_RPS // _ZR):
        pltpu.make_async_copy(
            zeros_hbm, acc.at[pl.ds(s * _RPS + t * _ZR, _ZR)], sem_s).wait()
    plsc.subcore_barrier()

    rows = (rows0, rows1)
    sem_g = (sem_g0, sem_g1)
    sem_i = (sem_i0, sem_i1)

    def step(j, rslot, dslot):
        orslot = 1 - rslot

        @pl.when(j >= 1)
        def _drain_prev_scatter():
            pltpu.make_async_copy(
                rows[orslot], acc.at[dring.at[(dslot + 3) % 4]],
                sem_s).wait()

        @pl.when(j + 1 < _CH)
        def _start_next_gather():
            pltpu.async_copy(table.at[src_v.at[pl.ds((j + 1) * _K, _K)]],
                             rows[orslot], sem_g[orslot])

        pltpu.make_async_copy(table.at[src_v.at[pl.ds(j * _K, _K)]],
                              rows[rslot], sem_g[rslot]).wait()

        @pl.when(j >= 1)
        def _wait_dst():
            pltpu.make_async_copy(edges.at[1, pl.ds(base + j * _K, _K)],
                                  dring.at[dslot], sem_i[rslot]).wait()

        pltpu.async_copy(rows[rslot], acc.at[dring.at[dslot]], sem_s,
                         add=True)

        @pl.when(j + 2 < _CH)
        def _start_next_dst():
            load_dst(j + 2, (dslot + 2) % 4, sem_i[rslot])

    def body(g, carry):
        for u in range(4):
            step(4 * g + u, u % 2, u)
        return carry

    lax.fori_loop(0, _CH // 4, body, 0)
    pltpu.make_async_copy(rows[1], acc.at[dring.at[3]], sem_s).wait()
    plsc.subcore_barrier()
    pltpu.sync_copy(acc.at[pl.ds(s * _RPS, _RPS)],
                    out.at[c, pl.ds(s * _RPS, _RPS)])


@functools.cache
def _make_gather_rows():
    mesh = plsc.VectorSubcoreMesh(core_axis_name="c", subcore_axis_name="s",
                                  num_cores=_NC, num_subcores=_NS)
    return functools.partial(
        pl.kernel,
        out_type=jax.ShapeDtypeStruct((3 * _B, _D), jnp.float32),
        mesh=mesh,
        scratch_types=[
            pltpu.VMEM((_GPW,), jnp.int32),
            pltpu.VMEM((_GPW, _D), jnp.float32),
            pltpu.SemaphoreType.DMA,
        ],
    )(_gather_rows_body)


def _gather_rows_body(table, idx, out, idx_v, rows_v, sem):
    c = lax.axis_index("c")
    s = lax.axis_index("s")
    base = (s * _NC + c) * _GPW
    pltpu.sync_copy(idx.at[pl.ds(base, _GPW)], idx_v)
    pltpu.async_copy(table.at[idx_v], rows_v, sem).wait()
    pltpu.sync_copy(rows_v, out.at[pl.ds(base, _GPW)])


def _mlp_block(eps_ref, x_ref, pa_ref, pb_ref, w1_ref, b1_ref, w2_ref,
               b2_ref, o_ref):
    z = (1.0 + eps_ref[0]) * x_ref[...] + pa_ref[0] + pb_ref[0]
    z = jnp.maximum(
        jnp.dot(z, w1_ref[...], preferred_element_type=jnp.float32)
        + b1_ref[...], 0.0)
    z = jnp.dot(z, w2_ref[...], preferred_element_type=jnp.float32) \
        + b2_ref[...]
    o_ref[...] = jnp.maximum(z, 0.0)


def _gin_mlp(x, parts, W1, b1, W2, b2, eps):
    R = 1000
    return pl.pallas_call(
        _mlp_block,
        grid=(_N // R,),
        in_specs=[
            pl.BlockSpec(memory_space=pltpu.SMEM),
            pl.BlockSpec((R, _D), lambda i: (i, 0)),
            pl.BlockSpec((1, R, _D), lambda i: (0, i, 0)),
            pl.BlockSpec((1, R, _D), lambda i: (1, i, 0)),
            pl.BlockSpec((_D, _D), lambda i: (0, 0)),
            pl.BlockSpec((1, _D), lambda i: (0, 0)),
            pl.BlockSpec((_D, _D), lambda i: (0, 0)),
            pl.BlockSpec((1, _D), lambda i: (0, 0)),
        ],
        out_specs=pl.BlockSpec((R, _D), lambda i: (i, 0)),
        out_shape=jax.ShapeDtypeStruct((_N, _D), jnp.float32),
    )(eps.reshape(1), x, parts, parts, W1, b1.reshape(1, _D), W2,
      b2.reshape(1, _D))


def _head_block(g_ref, p1_ref, pb1_ref, p2_ref, pb2_ref, o_ref):
    z = jnp.maximum(
        jnp.dot(g_ref[...], p1_ref[...], preferred_element_type=jnp.float32)
        + pb1_ref[...], 0.0)
    z = jnp.dot(z, p2_ref[...], preferred_element_type=jnp.float32) \
        + pb2_ref[...]
    sv = z[0:_B]
    sa = z[_B:2 * _B]
    sb = z[2 * _B:3 * _B]

    def cos(u, w):
        un = jnp.sqrt(jnp.sum(u * u, axis=1, keepdims=True))
        wn = jnp.sqrt(jnp.sum(w * w, axis=1, keepdims=True))
        return jnp.sum(u * w, axis=1, keepdims=True) / jnp.maximum(
            un * wn, 1e-8)

    pos = cos(sv, sa)
    neg = cos(sv, sb)
    num = jnp.exp(pos / _TAU)
    den = num + jnp.exp(neg / _TAU)
    o_ref[0, 0] = -jnp.sum(jnp.log(num / den)) / _B


def _head(g, P1, pb1, P2, pb2):
    return pl.pallas_call(
        _head_block,
        out_shape=jax.ShapeDtypeStruct((1, 1), jnp.float32),
        out_specs=pl.BlockSpec(memory_space=pltpu.SMEM),
    )(g, P1, pb1.reshape(1, _D), P2, pb2.reshape(1, _D))


def kernel(x, edge_index, v, a, b, W1_0, b1_0, W2_0, b2_0, eps_0, W1_1,
           b1_1, W2_1, b2_1, eps_1, P1, pb1, P2, pb2):
    pads = jnp.asarray(_PAD_EDGES)
    zrows = jnp.zeros((_ZR, _D), jnp.float32)

    segsum = _make_segment_sum()
    p0 = segsum(x, edge_index, pads, zrows)
    h = _gin_mlp(x, p0, W1_0, b1_0, W2_0, b2_0, eps_0)
    p1 = segsum(h, edge_index, pads, zrows)
    emb = _gin_mlp(h, p1, W1_1, b1_1, W2_1, b2_1, eps_1)

    idx = jnp.concatenate([v, a, b])
    g = _make_gather_rows()(emb, idx)
    loss = _head(g, P1, pb1, P2, pb2)
    return loss[0, 0]


# speedup vs baseline: 3.8072x; 1.0020x over previous
import functools

import jax
import jax.numpy as jnp
import numpy as np
from jax import lax
from jax.experimental import pallas as pl
from jax.experimental.pallas import tpu as pltpu
from jax.experimental.pallas import tpu_sc as plsc

_N = 10000
_D = 128
_E = 320000
_B = 1024
_TAU = 0.2

_NC = 2
_NS = 16
_NW = _NC * _NS
_K = 128
_CH = 80
_EPW = _CH * _K
_NPAD = 10240
_ZR = 128
_RPS = _NPAD // _NS
_GPW = 3 * _B // _NW
_PAD = _NW * _EPW - _E

_PAD_EDGES = np.stack([
    np.arange(_PAD, dtype=np.int32) % _N,
    _N + np.arange(_PAD, dtype=np.int32) % (_NPAD - _N),
])


@functools.cache
def _make_segment_sum():
    mesh = plsc.VectorSubcoreMesh(core_axis_name="c", subcore_axis_name="s",
                                  num_cores=_NC, num_subcores=_NS)
    return functools.partial(
        pl.kernel,
        out_type=jax.ShapeDtypeStruct((_NC, _NPAD, _D), jnp.float32),
        mesh=mesh,
        scratch_types=[
            pltpu.VMEM((_EPW,), jnp.int32),
            pltpu.VMEM((4, _K), jnp.int32),
            pltpu.VMEM((_K, _D), jnp.float32),
            pltpu.VMEM((_K, _D), jnp.float32),
            pltpu.VMEM_SHARED((_NPAD, _D), jnp.float32),
            pltpu.SemaphoreType.DMA,
            pltpu.SemaphoreType.DMA,
            pltpu.SemaphoreType.DMA,
            pltpu.SemaphoreType.DMA,
            pltpu.SemaphoreType.DMA,
        ],
    )(_segment_sum_body)


def _segment_sum_body(table, edges, pads, zeros_hbm, out, src_v, dring,
                      rows0, rows1, acc, sem_g0, sem_g1, sem_s, sem_i0,
                      sem_i1):
    c = lax.axis_index("c")
    s = lax.axis_index("s")
    wid = s * _NC + c
    base = wid * _EPW
    real31 = _E - (_NW - 1) * _EPW

    def load_dst(j, slot, sem):
        off = base + j * _K

        @pl.when(off + _K <= _E)
        def _real():
            pltpu.async_copy(edges.at[1, pl.ds(off, _K)], dring.at[slot],
                             sem)

        @pl.when(off >= _E)
        def _padded():
            pltpu.async_copy(pads.at[1, pl.ds(off - _E, _K)],
                             dring.at[slot], sem)

    @pl.when(wid < _NW - 1)
    def _src_full():
        pltpu.sync_copy(edges.at[0, pl.ds(base, _EPW)], src_v)

    @pl.when(wid == _NW - 1)
    def _src_split():
        pltpu.sync_copy(edges.at[0, pl.ds(base, real31)],
                        src_v.at[pl.ds(0, real31)])
        pltpu.sync_copy(pads.at[0], src_v.at[pl.ds(real31, _PAD)])

    def start_gather(j, buf, sem):
        pltpu.async_copy(table.at[src_v.at[pl.ds(j * _K, _K // 2)]],
                         buf.at[pl.ds(0, _K // 2)], sem)
        pltpu.async_copy(table.at[src_v.at[pl.ds(j * _K + _K // 2, _K // 2)]],
                         buf.at[pl.ds(_K // 2, _K // 2)], sem)

    def wait_gather(j, buf, sem):
        pltpu.make_async_copy(table.at[src_v.at[pl.ds(j * _K, _K // 2)]],
                              buf.at[pl.ds(0, _K // 2)], sem).wait()
        pltpu.make_async_copy(
            table.at[src_v.at[pl.ds(j * _K + _K // 2, _K // 2)]],
            buf.at[pl.ds(_K // 2, _K // 2)], sem).wait()

    load_dst(0, 0, sem_i0)
    pltpu.make_async_copy(edges.at[1, pl.ds(base, _K)], dring.at[0],
                          sem_i0).wait()
    start_gather(0, rows0, sem_g0)
    load_dst(1, 1, sem_i1)
    for t in range(_RPS // _ZR):
        pltpu.async_copy(zeros_hbm, acc.at[pl.ds(s * _RPS + t * _ZR, _ZR)],
                         sem_s)
    for t in range(_RPS // _ZR):
        pltpu.make_async_copy(
            zeros_hbm, acc.at[pl.ds(s * _RPS + t * _ZR, _ZR)], sem_s).wait()
    plsc.subcore_barrier()

    rows = (rows0, rows1)
    sem_g = (sem_g0, sem_g1)
    sem_i = (sem_i0, sem_i1)

    def step(j, rslot, dslot):
        orslot = 1 - rslot

        @pl.when(j >= 1)
        def _drain_prev_scatter():
            pltpu.make_async_copy(
                rows[orslot], acc.at[dring.at[(dslot + 3) % 4]],
                sem_s).wait()

        @pl.when(j + 1 < _CH)
        def _start_next_gather():
            start_gather(j + 1, rows[orslot], sem_g[orslot])

        wait_gather(j, rows[rslot], sem_g[rslot])

        @pl.when(j >= 1)
        def _wait_dst():
            pltpu.make_async_copy(edges.at[1, pl.ds(base + j * _K, _K)],
                                  dring.at[dslot], sem_i[rslot]).wait()

        pltpu.async_copy(rows[rslot], acc.at[dring.at[dslot]], sem_s,
                         add=True)

        @pl.when(j + 2 < _CH)
        def _start_next_dst():
            load_dst(j + 2, (dslot + 2) % 4, sem_i[rslot])

    def body(g, carry):
        for u in range(4):
            step(4 * g + u, u % 2, u)
        return carry

    lax.fori_loop(0, _CH // 4, body, 0)
    pltpu.make_async_copy(rows[1], acc.at[dring.at[3]], sem_s).wait()
    plsc.subcore_barrier()
    pltpu.sync_copy(acc.at[pl.ds(s * _RPS, _RPS)],
                    out.at[c, pl.ds(s * _RPS, _RPS)])


@functools.cache
def _make_gather_rows():
    mesh = plsc.VectorSubcoreMesh(core_axis_name="c", subcore_axis_name="s",
                                  num_cores=_NC, num_subcores=_NS)
    return functools.partial(
        pl.kernel,
        out_type=jax.ShapeDtypeStruct((3 * _B, _D), jnp.float32),
        mesh=mesh,
        scratch_types=[
            pltpu.VMEM((_GPW,), jnp.int32),
            pltpu.VMEM((_GPW, _D), jnp.float32),
            pltpu.SemaphoreType.DMA,
        ],
    )(_gather_rows_body)


def _gather_rows_body(table, idx, out, idx_v, rows_v, sem):
    c = lax.axis_index("c")
    s = lax.axis_index("s")
    base = (s * _NC + c) * _GPW
    pltpu.sync_copy(idx.at[pl.ds(base, _GPW)], idx_v)
    pltpu.async_copy(table.at[idx_v], rows_v, sem).wait()
    pltpu.sync_copy(rows_v, out.at[pl.ds(base, _GPW)])


def _mlp_block(eps_ref, x_ref, pa_ref, pb_ref, w1_ref, b1_ref, w2_ref,
               b2_ref, o_ref):
    z = (1.0 + eps_ref[0]) * x_ref[...] + pa_ref[0] + pb_ref[0]
    z = jnp.maximum(
        jnp.dot(z, w1_ref[...], preferred_element_type=jnp.float32)
        + b1_ref[...], 0.0)
    z = jnp.dot(z, w2_ref[...], preferred_element_type=jnp.float32) \
        + b2_ref[...]
    o_ref[...] = jnp.maximum(z, 0.0)


def _gin_mlp(x, parts, W1, b1, W2, b2, eps):
    R = 1000
    return pl.pallas_call(
        _mlp_block,
        grid=(_N // R,),
        in_specs=[
            pl.BlockSpec(memory_space=pltpu.SMEM),
            pl.BlockSpec((R, _D), lambda i: (i, 0)),
            pl.BlockSpec((1, R, _D), lambda i: (0, i, 0)),
            pl.BlockSpec((1, R, _D), lambda i: (1, i, 0)),
            pl.BlockSpec((_D, _D), lambda i: (0, 0)),
            pl.BlockSpec((1, _D), lambda i: (0, 0)),
            pl.BlockSpec((_D, _D), lambda i: (0, 0)),
            pl.BlockSpec((1, _D), lambda i: (0, 0)),
        ],
        out_specs=pl.BlockSpec((R, _D), lambda i: (i, 0)),
        out_shape=jax.ShapeDtypeStruct((_N, _D), jnp.float32),
    )(eps.reshape(1), x, parts, parts, W1, b1.reshape(1, _D), W2,
      b2.reshape(1, _D))


def _head_block(g_ref, p1_ref, pb1_ref, p2_ref, pb2_ref, o_ref):
    z = jnp.maximum(
        jnp.dot(g_ref[...], p1_ref[...], preferred_element_type=jnp.float32)
        + pb1_ref[...], 0.0)
    z = jnp.dot(z, p2_ref[...], preferred_element_type=jnp.float32) \
        + pb2_ref[...]
    sv = z[0:_B]
    sa = z[_B:2 * _B]
    sb = z[2 * _B:3 * _B]

    def cos(u, w):
        un = jnp.sqrt(jnp.sum(u * u, axis=1, keepdims=True))
        wn = jnp.sqrt(jnp.sum(w * w, axis=1, keepdims=True))
        return jnp.sum(u * w, axis=1, keepdims=True) / jnp.maximum(
            un * wn, 1e-8)

    pos = cos(sv, sa)
    neg = cos(sv, sb)
    num = jnp.exp(pos / _TAU)
    den = num + jnp.exp(neg / _TAU)
    o_ref[0, 0] = -jnp.sum(jnp.log(num / den)) / _B


def _head(g, P1, pb1, P2, pb2):
    return pl.pallas_call(
        _head_block,
        out_shape=jax.ShapeDtypeStruct((1, 1), jnp.float32),
        out_specs=pl.BlockSpec(memory_space=pltpu.SMEM),
    )(g, P1, pb1.reshape(1, _D), P2, pb2.reshape(1, _D))


def kernel(x, edge_index, v, a, b, W1_0, b1_0, W2_0, b2_0, eps_0, W1_1,
           b1_1, W2_1, b2_1, eps_1, P1, pb1, P2, pb2):
    pads = jnp.asarray(_PAD_EDGES)
    zrows = jnp.zeros((_ZR, _D), jnp.float32)

    segsum = _make_segment_sum()
    p0 = segsum(x, edge_index, pads, zrows)
    h = _gin_mlp(x, p0, W1_0, b1_0, W2_0, b2_0, eps_0)
    p1 = segsum(h, edge_index, pads, zrows)
    emb = _gin_mlp(h, p1, W1_1, b1_1, W2_1, b2_1, eps_1)

    idx = jnp.concatenate([v, a, b])
    g = _make_gather_rows()(emb, idx)
    loss = _head(g, P1, pb1, P2, pb2)
    return loss[0, 0]
